# baseline (speedup 1.0000x reference)
import functools

import jax
import jax.numpy as jnp
from jax import lax
from jax.experimental import pallas as pl
from jax.experimental.pallas import tpu as pltpu
from jax.experimental.pallas import tpu_sc as plsc

_NV = 100000
_E = 1600000
_NP = 2000
_NCLS = 6
_NW = 32

_NSPLIT = 25
_ES = _E // _NSPLIT
_EWS = _ES // _NW
_CH = 2000
_NCHS = _EWS // _CH
_NEV = _CH // 16
_UNROLL = 5

_mesh = plsc.VectorSubcoreMesh(core_axis_name="c", subcore_axis_name="s")
_sc_params = pltpu.CompilerParams(needs_layout_passes=False,
                                  use_tc_tiling_on_sc=False)


_NCHUNKS = _NSPLIT * _NCHS


@functools.partial(
    pl.kernel,
    mesh=_mesh,
    out_type=jax.ShapeDtypeStruct((_NW, 96), jnp.float32),
    scratch_types=[
        pltpu.VMEM((_NV,), jnp.float32),
        pltpu.VMEM((_NP,), jnp.int32),
        pltpu.VMEM((2, _CH), jnp.float32),
        pltpu.VMEM((2, _CH), jnp.int32),
        pltpu.VMEM((2, _CH), jnp.int32),
        pltpu.VMEM((96,), jnp.float32),
        pltpu.SemaphoreType.DMA,
        pltpu.SemaphoreType.DMA,
        pltpu.SemaphoreType.DMA,
    ],
    compiler_params=_sc_params,
)
def _edge_accum(*refs):
    area_hbm = refs[0]
    att_refs = refs[1:1 + _NSPLIT]
    (vox_hbm, prog_hbm, cls_hbm, out_hbm,
     area_t, cls_t, att_b, vox_b, prog_b, acc_b,
     sem_a, sem_v, sem_p) = refs[1 + _NSPLIT:]
    wid = lax.axis_index("s") * 2 + lax.axis_index("c")
    lane = lax.iota(jnp.int32, 16)
    zero = jnp.zeros((16,), jnp.float32)

    def fire(ci, slot):
        seg, j = divmod(ci, _NCHS)
        e0 = seg * _ES + (wid * _NCHS + j) * _CH
        pltpu.async_copy(att_refs[seg].at[wid * _NCHS + j],
                         att_b.at[slot], sem_a)
        pltpu.async_copy(vox_hbm.at[pl.ds(e0, _CH)], vox_b.at[slot], sem_v)
        pltpu.async_copy(prog_hbm.at[pl.ds(e0, _CH)], prog_b.at[slot], sem_p)

    def drain(ci, slot):
        seg, j = divmod(ci, _NCHS)
        e0 = seg * _ES + (wid * _NCHS + j) * _CH
        pltpu.make_async_copy(att_refs[seg].at[wid * _NCHS + j],
                              att_b.at[slot], sem_a).wait()
        pltpu.make_async_copy(vox_hbm.at[pl.ds(e0, _CH)], vox_b.at[slot],
                              sem_v).wait()
        pltpu.make_async_copy(prog_hbm.at[pl.ds(e0, _CH)], prog_b.at[slot],
                              sem_p).wait()

    fire(0, 0)
    pltpu.sync_copy(area_hbm, area_t)
    pltpu.sync_copy(cls_hbm, cls_t)
    for c6 in range(_NCLS):
        acc_b[pl.ds(c6 * 16, 16)] = zero

    def compute(slot):
        @plsc.parallel_loop(0, _NEV, unroll=_UNROLL)
        def _loop(v):
            o = v * 16
            vox = vox_b[slot, pl.ds(o, 16)]
            a = plsc.load_gather(area_t, [vox])
            val = att_b[slot, pl.ds(o, 16)] * a
            prog = prog_b[slot, pl.ds(o, 16)]
            cls = plsc.load_gather(cls_t, [prog])
            plsc.addupdate_scatter(acc_b, [cls * 16 + lane], val)

    for ci in range(_NCHUNKS):
        if ci + 1 < _NCHUNKS:
            fire(ci + 1, (ci + 1) % 2)
        drain(ci, ci % 2)
        compute(ci % 2)
    pltpu.sync_copy(acc_b, out_hbm.at[wid])


def _finish_body(part_ref, fv0_ref, fv1_ref, tgt_ref, tot_ref, adv_ref, tr_ref):
    adv = (-(jnp.sum(fv0_ref[...]) * (1.0 / 4096.0))
           - (jnp.sum(fv1_ref[...]) * (1.0 / 4096.0)))
    x = part_ref[...]
    s = [jnp.sum(x[:, c * 16:(c + 1) * 16]) for c in range(_NCLS)]
    tot_w = s[0] + s[1] + s[2] + s[3] + s[4] + s[5]
    inv = 1.0 / (tot_w + 1e-16)
    t = tgt_ref[...]
    losses = []
    for c in range(_NCLS):
        d = s[c] * inv - jnp.sum(t[:, c:c + 1])
        ad = jnp.abs(d)
        losses.append(jnp.where(ad < 1.0, 0.5 * d * d, ad - 0.5))
    tr = (losses[0] + losses[1] + losses[2] + losses[3] + losses[4]
          + losses[5]) * (1.0 / 6.0)
    tot = adv + tr
    tot_ref[...] = jnp.full((1, 1), tot, jnp.float32)
    adv_ref[...] = jnp.full((1, 1), adv, jnp.float32)
    tr_ref[...] = jnp.full((1, 1), tr, jnp.float32)


_finish = pl.pallas_call(
    _finish_body,
    out_shape=(jax.ShapeDtypeStruct((1, 1), jnp.float32),) * 3,
)


def kernel(fake_validity_voxel_0, fake_validity_voxel_1, fake_validity_program,
           voxel_feature, att, mask, program_target_ratio,
           pooled_program_feature_from_voxel, cross_edge_voxel_index,
           cross_edge_program_index, program_class_cluster,
           max_out_program_index, area_index_in_voxel_feature):
    area = jnp.take(voxel_feature, area_index_in_voxel_feature, axis=1)
    vox = cross_edge_voxel_index.astype(jnp.int32)
    prog = cross_edge_program_index.astype(jnp.int32)
    cluster = program_class_cluster.astype(jnp.int32)
    att_segs = [
        att[seg * _ES:(seg + 1) * _ES].reshape(_NW * _NCHS, _CH)
        for seg in range(_NSPLIT)
    ]
    partials = _edge_accum(area, *att_segs, vox, prog, cluster)
    tot, adv, tr = _finish(
        partials,
        fake_validity_voxel_0.reshape(32, 128),
        fake_validity_voxel_1.reshape(32, 128),
        program_target_ratio.reshape(1, _NCLS),
    )
    total_loss = tot.reshape(())
    adversarial_loss = adv.reshape(())
    target_ratio_loss = tr.reshape(())
    link_prediction_loss = jnp.zeros(())
    return (total_loss, adversarial_loss, link_prediction_loss,
            target_ratio_loss)


# speedup vs baseline: 1.1025x; 1.1025x over previous
import functools

import jax
import jax.numpy as jnp
from jax import lax
from jax.experimental import pallas as pl
from jax.experimental.pallas import tpu as pltpu
from jax.experimental.pallas import tpu_sc as plsc

_NV = 100000
_E = 1600000
_NP = 2000
_NCLS = 6
_NW = 32

_NSPLIT = 5
_ES = _E // _NSPLIT
_EWS = _ES // _NW
_CH = 2000
_NCHS = _EWS // _CH
_NEV = _CH // 16
_UNROLL = 5

_mesh = plsc.VectorSubcoreMesh(core_axis_name="c", subcore_axis_name="s")
_sc_params = pltpu.CompilerParams(needs_layout_passes=False,
                                  use_tc_tiling_on_sc=False)


_NCHUNKS = _NSPLIT * _NCHS


@functools.partial(
    pl.kernel,
    mesh=_mesh,
    out_type=jax.ShapeDtypeStruct((_NW, 96), jnp.float32),
    scratch_types=[
        pltpu.VMEM((_NV,), jnp.float32),
        pltpu.VMEM((_NP,), jnp.int32),
        pltpu.VMEM((4, _CH), jnp.float32),
        pltpu.VMEM((4, _CH), jnp.int32),
        pltpu.VMEM((4, _CH), jnp.int32),
        pltpu.VMEM((96,), jnp.float32),
        pltpu.SemaphoreType.DMA,
        pltpu.SemaphoreType.DMA,
        pltpu.SemaphoreType.DMA,
    ],
    compiler_params=_sc_params,
)
def _edge_accum(*refs):
    area_hbm = refs[0]
    att_refs = refs[1:1 + _NSPLIT]
    (vox_hbm, prog_hbm, cls_hbm, out_hbm,
     area_t, cls_t, att_b, vox_b, prog_b, acc_b,
     sem_a, sem_v, sem_p) = refs[1 + _NSPLIT:]
    wid = lax.axis_index("s") * 2 + lax.axis_index("c")
    lane = lax.iota(jnp.int32, 16)
    zero = jnp.zeros((16,), jnp.float32)

    def fire(ci, slot):
        seg, j = divmod(ci, _NCHS)
        e0 = seg * _ES + (wid * _NCHS + j) * _CH
        pltpu.async_copy(att_refs[seg].at[wid * _NCHS + j],
                         att_b.at[slot], sem_a)
        pltpu.async_copy(vox_hbm.at[pl.ds(e0, _CH)], vox_b.at[slot], sem_v)
        pltpu.async_copy(prog_hbm.at[pl.ds(e0, _CH)], prog_b.at[slot], sem_p)

    def drain(ci, slot):
        seg, j = divmod(ci, _NCHS)
        e0 = seg * _ES + (wid * _NCHS + j) * _CH
        pltpu.make_async_copy(att_refs[seg].at[wid * _NCHS + j],
                              att_b.at[slot], sem_a).wait()
        pltpu.make_async_copy(vox_hbm.at[pl.ds(e0, _CH)], vox_b.at[slot],
                              sem_v).wait()
        pltpu.make_async_copy(prog_hbm.at[pl.ds(e0, _CH)], prog_b.at[slot],
                              sem_p).wait()

    fire(0, 0)
    fire(1, 1)
    fire(2, 2)
    pltpu.sync_copy(area_hbm, area_t)
    pltpu.sync_copy(cls_hbm, cls_t)
    for c6 in range(_NCLS):
        acc_b[pl.ds(c6 * 16, 16)] = zero

    def compute(slot):
        @plsc.parallel_loop(0, _NEV, unroll=_UNROLL)
        def _loop(v):
            o = v * 16
            vox = vox_b[slot, pl.ds(o, 16)]
            a = plsc.load_gather(area_t, [vox])
            val = att_b[slot, pl.ds(o, 16)] * a
            prog = prog_b[slot, pl.ds(o, 16)]
            cls = plsc.load_gather(cls_t, [prog])
            plsc.addupdate_scatter(acc_b, [cls * 16 + lane], val)

    for ci in range(_NCHUNKS):
        if ci + 3 < _NCHUNKS:
            fire(ci + 3, (ci + 3) % 4)
        drain(ci, ci % 4)
        compute(ci % 4)
    pltpu.sync_copy(acc_b, out_hbm.at[wid])


def _finish_body(part_ref, fv0_ref, fv1_ref, tgt_ref, tot_ref, adv_ref, tr_ref):
    adv = (-(jnp.sum(fv0_ref[...]) * (1.0 / 4096.0))
           - (jnp.sum(fv1_ref[...]) * (1.0 / 4096.0)))
    x = part_ref[...]
    s = [jnp.sum(x[:, c * 16:(c + 1) * 16]) for c in range(_NCLS)]
    tot_w = s[0] + s[1] + s[2] + s[3] + s[4] + s[5]
    inv = 1.0 / (tot_w + 1e-16)
    t = tgt_ref[...]
    losses = []
    for c in range(_NCLS):
        d = s[c] * inv - jnp.sum(t[:, c:c + 1])
        ad = jnp.abs(d)
        losses.append(jnp.where(ad < 1.0, 0.5 * d * d, ad - 0.5))
    tr = (losses[0] + losses[1] + losses[2] + losses[3] + losses[4]
          + losses[5]) * (1.0 / 6.0)
    tot = adv + tr
    tot_ref[...] = jnp.full((1, 1), tot, jnp.float32)
    adv_ref[...] = jnp.full((1, 1), adv, jnp.float32)
    tr_ref[...] = jnp.full((1, 1), tr, jnp.float32)


_finish = pl.pallas_call(
    _finish_body,
    out_shape=(jax.ShapeDtypeStruct((1, 1), jnp.float32),) * 3,
)


def kernel(fake_validity_voxel_0, fake_validity_voxel_1, fake_validity_program,
           voxel_feature, att, mask, program_target_ratio,
           pooled_program_feature_from_voxel, cross_edge_voxel_index,
           cross_edge_program_index, program_class_cluster,
           max_out_program_index, area_index_in_voxel_feature):
    area = jnp.take(voxel_feature, area_index_in_voxel_feature, axis=1)
    vox = cross_edge_voxel_index.astype(jnp.int32)
    prog = cross_edge_program_index.astype(jnp.int32)
    cluster = program_class_cluster.astype(jnp.int32)
    att_segs = [
        att[seg * _ES:(seg + 1) * _ES].reshape(_NW * _NCHS, _CH)
        for seg in range(_NSPLIT)
    ]
    partials = _edge_accum(area, *att_segs, vox, prog, cluster)
    tot, adv, tr = _finish(
        partials,
        fake_validity_voxel_0.reshape(32, 128),
        fake_validity_voxel_1.reshape(32, 128),
        program_target_ratio.reshape(1, _NCLS),
    )
    total_loss = tot.reshape(())
    adversarial_loss = adv.reshape(())
    target_ratio_loss = tr.reshape(())
    link_prediction_loss = jnp.zeros(())
    return (total_loss, adversarial_loss, link_prediction_loss,
            target_ratio_loss)


# speedup vs baseline: 1.1232x; 1.0187x over previous
import functools

import jax
import jax.numpy as jnp
from jax import lax
from jax.experimental import pallas as pl
from jax.experimental.pallas import tpu as pltpu
from jax.experimental.pallas import tpu_sc as plsc

_NV = 100000
_E = 1600000
_NP = 2000
_NCLS = 6
_NW = 32

_NSPLIT = 5
_ES = _E // _NSPLIT
_EWS = _ES // _NW
_CH = 2000
_NCHS = _EWS // _CH
_NEV = _CH // 16
_UNROLL = 5

_mesh = plsc.VectorSubcoreMesh(core_axis_name="c", subcore_axis_name="s")
_sc_params = pltpu.CompilerParams(needs_layout_passes=False,
                                  use_tc_tiling_on_sc=False)


_NCHUNKS = _NSPLIT * _NCHS


@functools.partial(
    pl.kernel,
    mesh=_mesh,
    out_type=jax.ShapeDtypeStruct((_NW, 128), jnp.float32),
    scratch_types=[
        pltpu.VMEM((_NV,), jnp.float32),
        pltpu.VMEM((_NP,), jnp.int32),
        pltpu.VMEM((4, _CH), jnp.float32),
        pltpu.VMEM((4, _CH), jnp.int32),
        pltpu.VMEM((4, _CH), jnp.int32),
        pltpu.VMEM((128,), jnp.float32),
        pltpu.SemaphoreType.DMA,
        pltpu.SemaphoreType.DMA,
        pltpu.SemaphoreType.DMA,
    ],
    compiler_params=_sc_params,
)
def _edge_accum(*refs):
    area_hbm = refs[0]
    att_refs = refs[1:1 + _NSPLIT]
    (vox_hbm, prog_hbm, cls_hbm, out_hbm,
     area_t, cls_t, att_b, vox_b, prog_b, acc_b,
     sem_a, sem_v, sem_p) = refs[1 + _NSPLIT:]
    wid = lax.axis_index("s") * 2 + lax.axis_index("c")
    lane = lax.iota(jnp.int32, 16)
    zero = jnp.zeros((16,), jnp.float32)

    def fire(ci, slot):
        seg, j = divmod(ci, _NCHS)
        e0 = seg * _ES + (wid * _NCHS + j) * _CH
        pltpu.async_copy(att_refs[seg].at[wid * _NCHS + j],
                         att_b.at[slot], sem_a)
        pltpu.async_copy(vox_hbm.at[pl.ds(e0, _CH)], vox_b.at[slot], sem_v)
        pltpu.async_copy(prog_hbm.at[pl.ds(e0, _CH)], prog_b.at[slot], sem_p)

    def drain(ci, slot):
        seg, j = divmod(ci, _NCHS)
        e0 = seg * _ES + (wid * _NCHS + j) * _CH
        pltpu.make_async_copy(att_refs[seg].at[wid * _NCHS + j],
                              att_b.at[slot], sem_a).wait()
        pltpu.make_async_copy(vox_hbm.at[pl.ds(e0, _CH)], vox_b.at[slot],
                              sem_v).wait()
        pltpu.make_async_copy(prog_hbm.at[pl.ds(e0, _CH)], prog_b.at[slot],
                              sem_p).wait()

    fire(0, 0)
    fire(1, 1)
    fire(2, 2)
    pltpu.sync_copy(area_hbm, area_t)
    pltpu.sync_copy(cls_hbm, cls_t)
    for c6 in range(8):
        acc_b[pl.ds(c6 * 16, 16)] = zero

    def compute(slot):
        @plsc.parallel_loop(0, _NEV, unroll=_UNROLL)
        def _loop(v):
            o = v * 16
            vox = vox_b[slot, pl.ds(o, 16)]
            a = plsc.load_gather(area_t, [vox])
            val = att_b[slot, pl.ds(o, 16)] * a
            prog = prog_b[slot, pl.ds(o, 16)]
            cls = plsc.load_gather(cls_t, [prog])
            plsc.addupdate_scatter(acc_b, [cls * 16 + lane], val)

    for ci in range(_NCHUNKS):
        if ci + 3 < _NCHUNKS:
            fire(ci + 3, (ci + 3) % 4)
        drain(ci, ci % 4)
        compute(ci % 4)
    pltpu.sync_copy(acc_b, out_hbm.at[wid])


def _finish_body(part_ref, fv0_ref, fv1_ref, tgt_ref, tot_ref, adv_ref, tr_ref):
    adv = (-(jnp.sum(fv0_ref[...]) * (1.0 / 4096.0))
           - (jnp.sum(fv1_ref[...]) * (1.0 / 4096.0)))
    x = part_ref[...]
    s = [jnp.sum(x[:, c * 16:(c + 1) * 16]) for c in range(_NCLS)]
    tot_w = s[0] + s[1] + s[2] + s[3] + s[4] + s[5]
    inv = 1.0 / (tot_w + 1e-16)
    t = tgt_ref[...]
    losses = []
    for c in range(_NCLS):
        d = s[c] * inv - jnp.sum(t[:, c:c + 1])
        ad = jnp.abs(d)
        losses.append(jnp.where(ad < 1.0, 0.5 * d * d, ad - 0.5))
    tr = (losses[0] + losses[1] + losses[2] + losses[3] + losses[4]
          + losses[5]) * (1.0 / 6.0)
    tot = adv + tr
    tot_ref[...] = jnp.full((1, 1), tot, jnp.float32)
    adv_ref[...] = jnp.full((1, 1), adv, jnp.float32)
    tr_ref[...] = jnp.full((1, 1), tr, jnp.float32)


_finish = pl.pallas_call(
    _finish_body,
    out_shape=(jax.ShapeDtypeStruct((1, 1), jnp.float32),) * 3,
)


def kernel(fake_validity_voxel_0, fake_validity_voxel_1, fake_validity_program,
           voxel_feature, att, mask, program_target_ratio,
           pooled_program_feature_from_voxel, cross_edge_voxel_index,
           cross_edge_program_index, program_class_cluster,
           max_out_program_index, area_index_in_voxel_feature):
    area = jnp.take(voxel_feature, area_index_in_voxel_feature, axis=1)
    vox = cross_edge_voxel_index.astype(jnp.int32)
    prog = cross_edge_program_index.astype(jnp.int32)
    cluster = program_class_cluster.astype(jnp.int32)
    att_segs = [
        att[seg * _ES:(seg + 1) * _ES].reshape(_NW * _NCHS, _CH)
        for seg in range(_NSPLIT)
    ]
    partials = _edge_accum(area, *att_segs, vox, prog, cluster)
    tot, adv, tr = _finish(
        partials,
        fake_validity_voxel_0.reshape(32, 128),
        fake_validity_voxel_1.reshape(32, 128),
        program_target_ratio.reshape(1, _NCLS),
    )
    total_loss = tot.reshape(())
    adversarial_loss = adv.reshape(())
    target_ratio_loss = tr.reshape(())
    link_prediction_loss = jnp.zeros(())
    return (total_loss, adversarial_loss, link_prediction_loss,
            target_ratio_loss)
